# baseline (speedup 1.0000x reference)
import jax
import jax.numpy as jnp
from jax.experimental import pallas as pl
from jax.experimental.pallas import tpu as pltpu


def _router_body(x_ref, gate_ref, sel_ref, wts_ref, xbf_ref):
    x = x_ref[...]
    xbf_ref[...] = x.astype(jnp.bfloat16)
    seq_rep = jnp.mean(x, axis=1)
    logits = jax.lax.dot_general(
        seq_rep, gate_ref[...], (((1,), (1,)), ((), ())),
        preferred_element_type=jnp.float32)
    rw = jax.nn.softmax(logits, axis=-1)
    b, e = rw.shape
    col = jax.lax.broadcasted_iota(jnp.int32, (b, e), 1)
    v1 = jnp.max(rw, axis=-1, keepdims=True)
    i1 = jnp.min(jnp.where(rw == v1, col, e), axis=-1, keepdims=True)
    rw2 = jnp.where(col == i1, -jnp.inf, rw)
    v2 = jnp.max(rw2, axis=-1, keepdims=True)
    i2 = jnp.min(jnp.where(rw2 == v2, col, e), axis=-1, keepdims=True)
    total = v1 + v2
    sel_ref[...] = jnp.concatenate([i1, i2], axis=-1)
    wts_ref[...] = jnp.concatenate([v1 / total, v2 / total], axis=-1)


@jax.jit
def kernel(hidden_states, gate_w, w1, b1, w2, b2):
    bsz, seqlen, hdim = hidden_states.shape
    n_exp, ff_dim, _ = w1.shape
    top_k = 2

    sel, wts, xbf = pl.pallas_call(
        _router_body,
        out_shape=(
            jax.ShapeDtypeStruct((bsz, top_k), jnp.int32),
            jax.ShapeDtypeStruct((bsz, top_k), jnp.float32),
            jax.ShapeDtypeStruct((bsz, seqlen, hdim), jnp.bfloat16),
        ),
    )(hidden_states, gate_w)

    ff_blk = 512
    n_ff = ff_dim // ff_blk
    b1r = b1.reshape(n_exp * n_ff, 1, ff_blk)
    b2r = b2.reshape(n_exp, 1, hdim)

    grid_spec = pltpu.PrefetchScalarGridSpec(
        num_scalar_prefetch=2,
        grid=(bsz, n_ff),
        in_specs=[
            pl.BlockSpec((1, seqlen, hdim), lambda b, j, sel, wts: (b, 0, 0)),
            pl.BlockSpec((1, ff_blk, hdim),
                         lambda b, j, sel, wts: (sel[b, 0], j, 0)),
            pl.BlockSpec((1, ff_blk, hdim),
                         lambda b, j, sel, wts: (sel[b, 1], j, 0)),
            pl.BlockSpec((1, 1, ff_blk),
                         lambda b, j, sel, wts: (sel[b, 0] * n_ff + j, 0, 0)),
            pl.BlockSpec((1, 1, ff_blk),
                         lambda b, j, sel, wts: (sel[b, 1] * n_ff + j, 0, 0)),
            pl.BlockSpec((1, hdim, ff_blk),
                         lambda b, j, sel, wts: (sel[b, 0], 0, j)),
            pl.BlockSpec((1, hdim, ff_blk),
                         lambda b, j, sel, wts: (sel[b, 1], 0, j)),
            pl.BlockSpec((1, 1, hdim),
                         lambda b, j, sel, wts: (sel[b, 0], 0, 0)),
            pl.BlockSpec((1, 1, hdim),
                         lambda b, j, sel, wts: (sel[b, 1], 0, 0)),
        ],
        out_specs=pl.BlockSpec((1, seqlen, hdim),
                               lambda b, j, sel, wts: (b, 0, 0)),
    )

    def body2(sel_ref, wts_ref, x_ref, w1a_ref, w1b_ref, b1a_ref, b1b_ref,
              w2a_ref, w2b_ref, b2a_ref, b2b_ref, out_ref):
        b = pl.program_id(0)
        j = pl.program_id(1)
        x = x_ref[0]

        def branch(w1_ref, b1_ref, w2_ref, k):
            w = wts_ref[b, k]
            h = jax.lax.dot_general(
                x, w1_ref[0].astype(jnp.bfloat16), (((1,), (1,)), ((), ())),
                preferred_element_type=jnp.float32)
            h = h + b1_ref[0, 0][None, :]
            h = (0.5 * w) * h * (1.0 + jax.lax.erf(h * 0.7071067811865476))
            return jax.lax.dot_general(
                h.astype(jnp.bfloat16), w2_ref[0].astype(jnp.bfloat16),
                (((1,), (1,)), ((), ())),
                preferred_element_type=jnp.float32)

        contrib = branch(w1a_ref, b1a_ref, w2a_ref, 0)
        contrib += branch(w1b_ref, b1b_ref, w2b_ref, 1)

        @pl.when(j == 0)
        def _init():
            out_ref[0] = (wts_ref[b, 0] * b2a_ref[0, 0][None, :]
                          + wts_ref[b, 1] * b2b_ref[0, 0][None, :] + contrib)

        @pl.when(j > 0)
        def _acc():
            out_ref[0] += contrib

    out = pl.pallas_call(
        body2,
        grid_spec=grid_spec,
        out_shape=jax.ShapeDtypeStruct((bsz, seqlen, hdim), jnp.float32),
        compiler_params=pltpu.CompilerParams(
            dimension_semantics=("parallel", "arbitrary"),
            vmem_limit_bytes=100 * 1024 * 1024,
        ),
    )(sel, wts, xbf, w1, w1, b1r, b1r, w2, w2, b2r, b2r)
    return out


# speedup vs baseline: 1.0489x; 1.0489x over previous
import jax
import jax.numpy as jnp
from jax.experimental import pallas as pl
from jax.experimental.pallas import tpu as pltpu


def _router_body(x_ref, gate_ref, sel_ref, wts_ref):
    x = x_ref[...]
    seq_rep = jnp.mean(x, axis=1)
    logits = jax.lax.dot_general(
        seq_rep, gate_ref[...], (((1,), (1,)), ((), ())),
        preferred_element_type=jnp.float32)
    rw = jax.nn.softmax(logits, axis=-1)
    b, e = rw.shape
    col = jax.lax.broadcasted_iota(jnp.int32, (b, e), 1)
    v1 = jnp.max(rw, axis=-1, keepdims=True)
    i1 = jnp.min(jnp.where(rw == v1, col, e), axis=-1, keepdims=True)
    rw2 = jnp.where(col == i1, -jnp.inf, rw)
    v2 = jnp.max(rw2, axis=-1, keepdims=True)
    i2 = jnp.min(jnp.where(rw2 == v2, col, e), axis=-1, keepdims=True)
    total = v1 + v2
    sel_ref[...] = jnp.concatenate([i1, i2], axis=-1)
    wts_ref[...] = jnp.concatenate([v1 / total, v2 / total], axis=-1)


@jax.jit
def kernel(hidden_states, gate_w, w1, b1, w2, b2):
    bsz, seqlen, hdim = hidden_states.shape
    n_exp, ff_dim, _ = w1.shape
    top_k = 2

    sel, wts = pl.pallas_call(
        _router_body,
        out_shape=(
            jax.ShapeDtypeStruct((bsz, top_k), jnp.int32),
            jax.ShapeDtypeStruct((bsz, top_k), jnp.float32),
        ),
    )(hidden_states, gate_w)

    ff_blk = 512
    n_ff = ff_dim // ff_blk
    b1r = b1.reshape(n_exp * n_ff, 1, ff_blk)
    b2r = b2.reshape(n_exp, 1, hdim)

    grid_spec = pltpu.PrefetchScalarGridSpec(
        num_scalar_prefetch=2,
        grid=(bsz, n_ff),
        in_specs=[
            pl.BlockSpec((1, seqlen, hdim), lambda b, j, sel, wts: (b, 0, 0)),
            pl.BlockSpec((1, ff_blk, hdim),
                         lambda b, j, sel, wts: (sel[b, 0], j, 0)),
            pl.BlockSpec((1, ff_blk, hdim),
                         lambda b, j, sel, wts: (sel[b, 1], j, 0)),
            pl.BlockSpec((1, 1, ff_blk),
                         lambda b, j, sel, wts: (sel[b, 0] * n_ff + j, 0, 0)),
            pl.BlockSpec((1, 1, ff_blk),
                         lambda b, j, sel, wts: (sel[b, 1] * n_ff + j, 0, 0)),
            pl.BlockSpec((1, hdim, ff_blk),
                         lambda b, j, sel, wts: (sel[b, 0], 0, j)),
            pl.BlockSpec((1, hdim, ff_blk),
                         lambda b, j, sel, wts: (sel[b, 1], 0, j)),
            pl.BlockSpec((1, 1, hdim),
                         lambda b, j, sel, wts: (sel[b, 0], 0, 0)),
            pl.BlockSpec((1, 1, hdim),
                         lambda b, j, sel, wts: (sel[b, 1], 0, 0)),
        ],
        out_specs=pl.BlockSpec((1, seqlen, hdim),
                               lambda b, j, sel, wts: (b, 0, 0)),
    )

    def body2(sel_ref, wts_ref, x_ref, w1a_ref, w1b_ref, b1a_ref, b1b_ref,
              w2a_ref, w2b_ref, b2a_ref, b2b_ref, out_ref):
        b = pl.program_id(0)
        j = pl.program_id(1)
        x = x_ref[0]

        def branch(w1_ref, b1_ref, w2_ref, k):
            w = wts_ref[b, k]
            h = jax.lax.dot_general(
                x, w1_ref[0], (((1,), (1,)), ((), ())),
                preferred_element_type=jnp.float32)
            h = h + b1_ref[0, 0][None, :]
            h = (0.5 * w) * h * (1.0 + jax.lax.erf(h * 0.7071067811865476))
            return jax.lax.dot_general(
                h, w2_ref[0], (((1,), (1,)), ((), ())),
                preferred_element_type=jnp.float32)

        contrib = branch(w1a_ref, b1a_ref, w2a_ref, 0)
        contrib += branch(w1b_ref, b1b_ref, w2b_ref, 1)

        @pl.when(j == 0)
        def _init():
            out_ref[0] = (wts_ref[b, 0] * b2a_ref[0, 0][None, :]
                          + wts_ref[b, 1] * b2b_ref[0, 0][None, :] + contrib)

        @pl.when(j > 0)
        def _acc():
            out_ref[0] += contrib

    out = pl.pallas_call(
        body2,
        grid_spec=grid_spec,
        out_shape=jax.ShapeDtypeStruct((bsz, seqlen, hdim), jnp.float32),
        compiler_params=pltpu.CompilerParams(
            dimension_semantics=("parallel", "arbitrary"),
            vmem_limit_bytes=100 * 1024 * 1024,
        ),
    )(sel, wts, hidden_states, w1, w1, b1r, b1r, w2, w2, b2r, b2r)
    return out
